# baseline (speedup 1.0000x reference)
import jax
import jax.numpy as jnp
from jax.experimental import pallas as pl
from jax.experimental.pallas import tpu as pltpu

_VG = 8


def _ctg_body(x_ref, w_ref, b_ref, m_ref, o_ref):
    bn, c_in, v, tt = x_ref.shape
    c_out = o_ref.shape[1]
    for i in range(bn):
        for h in range(v // _VG):
            lo = h * _VG
            xh = x_ref[i, :, lo:lo + _VG, :].reshape(c_in * _VG, tt)
            acc = jax.lax.dot_general(
                w_ref[...], xh,
                dimension_numbers=(((1,), (0,)), ((), ())),
                preferred_element_type=jnp.float32)
            acc3 = acc.reshape(c_out, _VG, tt)
            o_ref[i, :, lo:lo + _VG, :] = (
                (acc3 + b_ref[...]) * m_ref[i, lo:lo + _VG, :])

def _block_diag_weight(w2, vg):
    c_out, c_in = w2.shape
    w_r = jnp.broadcast_to(w2[:, None, :], (c_out, vg, c_in))
    w_r = w_r.reshape(c_out * vg, c_in)
    col = jnp.arange(c_in * vg)
    q = (jnp.arange(c_in)[:, None] == (col[None, :] // vg)).astype(w2.dtype)
    m = ((jnp.arange(c_out * vg)[:, None] % vg) == (col[None, :] % vg))
    return (w_r @ q) * m.astype(w2.dtype)


def kernel(x, A, weight, bias, mask, *, t_tile=128, bn=8):
    N, C_in, T, V = x.shape
    C_out = weight.shape[0]
    if T % t_tile != 0:
        t_tile = T
    Tt = t_tile
    if N % bn != 0:
        bn = 1
    grid = (N // bn, T // Tt)
    assert V % _VG == 0, "node count must be a multiple of the sublane tile"

    xp = x.transpose(0, 1, 3, 2)
    mp = mask.transpose(0, 2, 1)

    w2 = weight.reshape(C_out, C_in)
    w_bd = _block_diag_weight(w2, _VG)
    b3 = bias.reshape(C_out, 1, 1).astype(jnp.float32)

    outp = pl.pallas_call(
        _ctg_body,
        out_shape=jax.ShapeDtypeStruct((N, C_out, V, T), x.dtype),
        grid=grid,
        in_specs=[
            pl.BlockSpec((bn, C_in, V, Tt), lambda n, s: (n, 0, 0, s)),
            pl.BlockSpec((C_out * _VG, C_in * _VG), lambda n, s: (0, 0)),
            pl.BlockSpec((C_out, 1, 1), lambda n, s: (0, 0, 0)),
            pl.BlockSpec((bn, V, Tt), lambda n, s: (n, 0, s)),
        ],
        out_specs=pl.BlockSpec((bn, C_out, V, Tt), lambda n, s: (n, 0, 0, s)),
        compiler_params=pltpu.CompilerParams(
            dimension_semantics=("parallel", "parallel")),
        cost_estimate=pl.CostEstimate(
            flops=2 * N * C_out * C_in * T * V,
            transcendentals=0,
            bytes_accessed=4 * (N * C_in * T * V + N * C_out * T * V + N * T * V)),
    )(xp, w_bd, b3, mp)

    return outp.transpose(0, 1, 3, 2), A


# speedup vs baseline: 1.1997x; 1.1997x over previous
import jax
import jax.numpy as jnp
from jax.experimental import pallas as pl
from jax.experimental.pallas import tpu as pltpu

_VG = 8


def _ctg_body(x_ref, w_ref, b_ref, m_ref, o_ref):
    bn, c_in, v, tt = x_ref.shape
    c_out = o_ref.shape[1]
    for i in range(bn):
        for h in range(v // _VG):
            lo = h * _VG
            xh = x_ref[i, :, lo:lo + _VG, :].reshape(c_in * _VG, tt)
            acc = jax.lax.dot_general(
                w_ref[...], xh,
                dimension_numbers=(((1,), (0,)), ((), ())),
                preferred_element_type=jnp.float32)
            acc3 = acc.reshape(c_out, _VG, tt)
            o_ref[i, :, lo:lo + _VG, :] = (
                (acc3 + b_ref[...]) * m_ref[i, lo:lo + _VG, :])


def _block_diag_weight(w2, vg):
    c_out, c_in = w2.shape
    w_r = jnp.broadcast_to(w2[:, None, :], (c_out, vg, c_in))
    w_r = w_r.reshape(c_out * vg, c_in)
    col = jnp.arange(c_in * vg)
    q = (jnp.arange(c_in)[:, None] == (col[None, :] // vg)).astype(w2.dtype)
    m = ((jnp.arange(c_out * vg)[:, None] % vg) == (col[None, :] % vg))
    return (w_r @ q) * m.astype(w2.dtype)


def kernel(x, A, weight, bias, mask, *, t_tile=256, bn=8):
    N, C_in, T, V = x.shape
    C_out = weight.shape[0]
    if T % t_tile != 0:
        t_tile = T
    Tt = t_tile
    if N % bn != 0:
        bn = 1
    grid = (N // bn, T // Tt)
    assert V % _VG == 0, "node count must be a multiple of the sublane tile"

    xp = x.transpose(0, 1, 3, 2)
    mp = mask.transpose(0, 2, 1)

    w2 = weight.reshape(C_out, C_in)
    w_bd = _block_diag_weight(w2, _VG)
    b3 = bias.reshape(C_out, 1, 1).astype(jnp.float32)

    outp = pl.pallas_call(
        _ctg_body,
        out_shape=jax.ShapeDtypeStruct((N, C_out, V, T), x.dtype),
        grid=grid,
        in_specs=[
            pl.BlockSpec((bn, C_in, V, Tt), lambda n, s: (n, 0, 0, s)),
            pl.BlockSpec((C_out * _VG, C_in * _VG), lambda n, s: (0, 0)),
            pl.BlockSpec((C_out, 1, 1), lambda n, s: (0, 0, 0)),
            pl.BlockSpec((bn, V, Tt), lambda n, s: (n, 0, s)),
        ],
        out_specs=pl.BlockSpec((bn, C_out, V, Tt), lambda n, s: (n, 0, 0, s)),
        compiler_params=pltpu.CompilerParams(
            dimension_semantics=("parallel", "parallel")),
        cost_estimate=pl.CostEstimate(
            flops=2 * N * C_out * C_in * T * V,
            transcendentals=0,
            bytes_accessed=4 * (N * C_in * T * V + N * C_out * T * V + N * T * V)),
    )(xp, w_bd, b3, mp)

    return outp.transpose(0, 1, 3, 2), A
